# baseline (speedup 1.0000x reference)
import functools

import jax
import jax.numpy as jnp
from jax import lax
from jax.experimental import pallas as pl
from jax.experimental.pallas import tpu as pltpu
from jax.experimental.pallas import tpu_sc as plsc

_BINS = 10
_NW = 32
_ROWS_MAIN = 3128
_CHUNK = 184
_TAIL = 88

_SP_COEF = (
    0.6931471806874083, -0.5000000114525803, 0.1250002509859619,
    -2.327675999733397e-06, -0.00519710417569212, -3.087543114007119e-05,
    0.00039693630143537175, -4.513049959535683e-05, -7.231349271430002e-06,
)


def _sc_body(pred_hbm, target_hbm, out_hbm, x_v, t_v, acc_v, cnt_v, res_v):
    wid = lax.axis_index("s") * 2 + lax.axis_index("c")
    base = wid * _ROWS_MAIN

    zeros16 = jnp.zeros((16,), jnp.float32)
    ones16 = jnp.ones((16,), jnp.float32)
    lanes = lax.iota(jnp.int32, 16)
    for i in range(_BINS):
        acc_v[pl.ds(i * 16, 16)] = zeros16
        cnt_v[pl.ds(i * 16, 16)] = zeros16

    def process_chunk(row0, nrows):
        pltpu.sync_copy(pred_hbm.at[pl.ds(row0, nrows), :], x_v.at[pl.ds(0, nrows), :])
        pltpu.sync_copy(target_hbm.at[pl.ds(row0, nrows), :], t_v.at[pl.ds(0, nrows), :])

        def row_body(r, carry):
            for cidx in range(5):
                x = x_v[r, pl.ds(cidx * 16, 16)]
                t = t_v[r, pl.ds(cidx * 16, 16)]
                p = 1.0 / (1.0 + jnp.exp(-x))
                sp = jnp.full((16,), _SP_COEF[-1], jnp.float32)
                for coef in _SP_COEF[-2::-1]:
                    sp = sp * p + coef
                bce = p * (1.0 - t) + sp
                g = jnp.abs(p - t)
                b = jnp.minimum((g * jnp.float32(_BINS)).astype(jnp.int32), _BINS - 1)
                idx = b * 16 + lanes
                plsc.addupdate_scatter(acc_v, [idx], bce)
                plsc.addupdate_scatter(cnt_v, [idx], ones16)
            return carry

        lax.fori_loop(0, nrows, row_body, 0)

    def chunk_body(ci, carry):
        process_chunk(base + ci * _CHUNK, _CHUNK)
        return carry

    lax.fori_loop(0, 16, chunk_body, 0)

    @pl.when(wid < _NW - 1)
    def _last_main_chunk():
        process_chunk(base + 16 * _CHUNK, _CHUNK)

    @pl.when(wid == _NW - 1)
    def _tail_chunk():
        process_chunk(base + 16 * _CHUNK, _TAIL)

    for i in range(_BINS):
        res_v[i // 8, pl.ds((i % 8) * 16, 16)] = acc_v[pl.ds(i * 16, 16)]
        res_v[2 + i // 8, pl.ds((i % 8) * 16, 16)] = cnt_v[pl.ds(i * 16, 16)]
    pltpu.sync_copy(res_v, out_hbm.at[wid])


def _combine_kernel(parts_ref, out_ref):
    v = jnp.sum(parts_ref[...], axis=0)
    m = jnp.sum(v.reshape(8, 8, 16), axis=2)
    loss = jnp.float32(0.0)
    nbins = jnp.float32(0.0)
    for b in range(_BINS):
        s = m[b // 8, b % 8]
        cnt = m[2 + b // 8, b % 8]
        nonempty = cnt > 0.0
        loss += jnp.where(nonempty, s / jnp.maximum(cnt, 1.0), 0.0)
        nbins += nonempty.astype(jnp.float32)
    out_ref[0, 0] = loss / jnp.maximum(nbins, 1.0)


@jax.jit
def kernel(pred, target, label_weight):
    mesh = plsc.VectorSubcoreMesh(core_axis_name="c", subcore_axis_name="s")
    sc_hist = pl.kernel(
        _sc_body,
        out_type=jax.ShapeDtypeStruct((_NW, 8, 128), jnp.float32),
        mesh=mesh,
        scratch_types=[
            pltpu.VMEM((_CHUNK, 80), jnp.float32),
            pltpu.VMEM((_CHUNK, 80), jnp.float32),
            pltpu.VMEM((16 * _BINS,), jnp.float32),
            pltpu.VMEM((16 * _BINS,), jnp.float32),
            pltpu.VMEM((8, 128), jnp.float32),
        ],
        compiler_params=pltpu.CompilerParams(
            use_tc_tiling_on_sc=True, needs_layout_passes=False
        ),
    )
    parts = sc_hist(pred, target)
    out = pl.pallas_call(
        _combine_kernel,
        out_specs=pl.BlockSpec(memory_space=pltpu.SMEM),
        out_shape=jax.ShapeDtypeStruct((1, 1), jnp.float32),
    )(parts)
    return out[0, 0]


# speedup vs baseline: 1.2730x; 1.2730x over previous
import functools

import jax
import jax.numpy as jnp
from jax import lax
from jax.experimental import pallas as pl
from jax.experimental.pallas import tpu as pltpu
from jax.experimental.pallas import tpu_sc as plsc

_BINS = 10
_NW = 32
_ROWS_MAIN = 3128
_CHUNK = 184
_TAIL = 88

_SP_COEF = (
    0.6931471806874083, -0.5000000114525803, 0.1250002509859619,
    -2.327675999733397e-06, -0.00519710417569212, -3.087543114007119e-05,
    0.00039693630143537175, -4.513049959535683e-05, -7.231349271430002e-06,
)


def _sc_body(pred_hbm, target_hbm, out_hbm, x_v, t_v, acc_v, cnt_v, res_v):
    wid = lax.axis_index("s") * 2 + lax.axis_index("c")
    base = wid * _ROWS_MAIN

    zeros16 = jnp.zeros((16,), jnp.float32)
    ones16 = jnp.ones((16,), jnp.float32)
    lanes = lax.iota(jnp.int32, 16)
    for i in range(_BINS):
        acc_v[pl.ds(i * 16, 16)] = zeros16
        cnt_v[pl.ds(i * 16, 16)] = zeros16

    def process_chunk(row0, nrows):
        pltpu.sync_copy(pred_hbm.at[pl.ds(row0, nrows), :], x_v.at[pl.ds(0, nrows), :])
        pltpu.sync_copy(target_hbm.at[pl.ds(row0, nrows), :], t_v.at[pl.ds(0, nrows), :])

        c = _SP_COEF

        def row_body(r4, carry):
            for dr in range(4):
                r = r4 * 4 + dr
                for cidx in range(5):
                    x = x_v[r, pl.ds(cidx * 16, 16)]
                    t = t_v[r, pl.ds(cidx * 16, 16)]
                    p = 1.0 / (1.0 + jnp.exp(-x))
                    p2 = p * p
                    p4 = p2 * p2
                    q0 = (c[0] + c[1] * p) + p2 * (c[2] + c[3] * p)
                    q1 = (c[4] + c[5] * p) + p2 * (c[6] + c[7] * p)
                    sp = q0 + p4 * (q1 + p4 * c[8])
                    bce = p * (1.0 - t) + sp
                    g = jnp.abs(p - t)
                    b = jnp.minimum((g * jnp.float32(_BINS)).astype(jnp.int32), _BINS - 1)
                    idx = b * 16 + lanes
                    plsc.addupdate_scatter(acc_v, [idx], bce)
                    plsc.addupdate_scatter(cnt_v, [idx], ones16)
            return carry

        lax.fori_loop(0, nrows // 4, row_body, 0)

    def chunk_body(ci, carry):
        process_chunk(base + ci * _CHUNK, _CHUNK)
        return carry

    lax.fori_loop(0, 16, chunk_body, 0)

    @pl.when(wid < _NW - 1)
    def _last_main_chunk():
        process_chunk(base + 16 * _CHUNK, _CHUNK)

    @pl.when(wid == _NW - 1)
    def _tail_chunk():
        process_chunk(base + 16 * _CHUNK, _TAIL)

    for i in range(_BINS):
        res_v[i // 8, pl.ds((i % 8) * 16, 16)] = acc_v[pl.ds(i * 16, 16)]
        res_v[2 + i // 8, pl.ds((i % 8) * 16, 16)] = cnt_v[pl.ds(i * 16, 16)]
    pltpu.sync_copy(res_v, out_hbm.at[wid])


def _combine_kernel(parts_ref, out_ref):
    v = jnp.sum(parts_ref[...], axis=0)
    m = jnp.sum(v.reshape(8, 8, 16), axis=2)
    loss = jnp.float32(0.0)
    nbins = jnp.float32(0.0)
    for b in range(_BINS):
        s = m[b // 8, b % 8]
        cnt = m[2 + b // 8, b % 8]
        nonempty = cnt > 0.0
        loss += jnp.where(nonempty, s / jnp.maximum(cnt, 1.0), 0.0)
        nbins += nonempty.astype(jnp.float32)
    out_ref[0, 0] = loss / jnp.maximum(nbins, 1.0)


@jax.jit
def kernel(pred, target, label_weight):
    mesh = plsc.VectorSubcoreMesh(core_axis_name="c", subcore_axis_name="s")
    sc_hist = pl.kernel(
        _sc_body,
        out_type=jax.ShapeDtypeStruct((_NW, 8, 128), jnp.float32),
        mesh=mesh,
        scratch_types=[
            pltpu.VMEM((_CHUNK, 80), jnp.float32),
            pltpu.VMEM((_CHUNK, 80), jnp.float32),
            pltpu.VMEM((16 * _BINS,), jnp.float32),
            pltpu.VMEM((16 * _BINS,), jnp.float32),
            pltpu.VMEM((8, 128), jnp.float32),
        ],
        compiler_params=pltpu.CompilerParams(
            use_tc_tiling_on_sc=True, needs_layout_passes=False
        ),
    )
    parts = sc_hist(pred, target)
    out = pl.pallas_call(
        _combine_kernel,
        out_specs=pl.BlockSpec(memory_space=pltpu.SMEM),
        out_shape=jax.ShapeDtypeStruct((1, 1), jnp.float32),
    )(parts)
    return out[0, 0]


# speedup vs baseline: 2.8412x; 2.2319x over previous
import functools

import jax
import jax.numpy as jnp
from jax.experimental import pallas as pl
from jax.experimental.pallas import tpu as pltpu

_BINS = 10


def _ghm_kernel(pred_a_ref, pred_b_ref, target_a_ref, target_b_ref, out_ref,
                acc_ref, *, nsteps):
    step = pl.program_id(0)

    @pl.when(step == 0)
    def _init():
        acc_ref[...] = jnp.zeros_like(acc_ref)

    bn, c = pred_a_ref.shape
    for x_ref, t_ref in ((pred_a_ref, target_a_ref), (pred_b_ref, target_b_ref)):
        x = x_ref[...].reshape(bn // 8, 8, c)
        t = t_ref[...].reshape(bn // 8, 8, c)
        p = jax.nn.sigmoid(x)
        bce = p * (1.0 - t) + jnp.log1p(jnp.exp(-p))
        g = jnp.abs(p - t)

        acc_ref[0] += jnp.sum(bce, axis=0)
        for k in range(1, _BINS):
            ge = (g >= jnp.float32(k) / jnp.float32(_BINS)).astype(jnp.float32)
            acc_ref[2 * k] += jnp.sum(ge * bce, axis=0)
            acc_ref[2 * k + 1] += jnp.sum(ge, axis=0)

    @pl.when(step == nsteps - 1)
    def _finish():
        total = jnp.float32(2 * nsteps) * bn * c
        t_cum = [jnp.sum(acc_ref[0])]
        c_cum = [total]
        for k in range(1, _BINS):
            t_cum.append(jnp.sum(acc_ref[2 * k]))
            c_cum.append(jnp.sum(acc_ref[2 * k + 1]))
        t_cum.append(jnp.float32(0.0))
        c_cum.append(jnp.float32(0.0))
        acc = jnp.float32(0.0)
        nbins = jnp.float32(0.0)
        for b in range(_BINS):
            cnt = c_cum[b] - c_cum[b + 1]
            s = t_cum[b] - t_cum[b + 1]
            nonempty = cnt > 0.0
            acc += jnp.where(nonempty, s / jnp.maximum(cnt, 1.0), 0.0)
            nbins += nonempty.astype(jnp.float32)
        out_ref[0, 0] = acc / jnp.maximum(nbins, 1.0)


@jax.jit
def kernel(pred, target, label_weight):
    n, c = pred.shape
    block_n = 5000
    nsteps = n // (2 * block_n)
    out = pl.pallas_call(
        functools.partial(_ghm_kernel, nsteps=nsteps),
        grid=(nsteps,),
        in_specs=[
            pl.BlockSpec((block_n, c), lambda i: (2 * i, 0)),
            pl.BlockSpec((block_n, c), lambda i: (2 * i + 1, 0)),
            pl.BlockSpec((block_n, c), lambda i: (2 * i, 0)),
            pl.BlockSpec((block_n, c), lambda i: (2 * i + 1, 0)),
        ],
        out_specs=pl.BlockSpec(memory_space=pltpu.SMEM),
        out_shape=jax.ShapeDtypeStruct((1, 1), jnp.float32),
        scratch_shapes=[
            pltpu.VMEM((2 * _BINS, 8, c), jnp.float32),
        ],
    )(pred, pred, target, target)
    return out[0, 0]
